# baseline (speedup 1.0000x reference)
import functools

import jax
import jax.numpy as jnp
from jax import lax
from jax.experimental import pallas as pl
from jax.experimental.pallas import tpu as pltpu
from jax.experimental.pallas import tpu_sc as plsc

_B = 16384
_N = 1000
_D = 128

_K = 2
_BCH = _B // _K

_NC = 2
_NS = 16
_NW = _NC * _NS
_BPW = _BCH // _NW
_CH = 128
_NCH = _BPW // _CH

_BM = 4096


def _argmax_body(x_ref, idx_ref):
    xb = x_ref[...]
    idx_ref[...] = jnp.argmax(xb, axis=1).astype(jnp.int32)


def _argmax_chunk(x, k):
    return pl.pallas_call(
        _argmax_body,
        grid=(_BCH // _BM,),
        in_specs=[pl.BlockSpec((_BM, _N), lambda i, k=k: (k * (_BCH // _BM) + i, 0))],
        out_specs=pl.BlockSpec((_BM,), lambda i: (i,)),
        out_shape=jax.ShapeDtypeStruct((_BCH,), jnp.int32),
    )(x)


@functools.cache
def _gather_sc():
    mesh = plsc.VectorSubcoreMesh(core_axis_name="c", subcore_axis_name="s")

    @functools.partial(
        pl.kernel,
        mesh=mesh,
        out_type=jax.ShapeDtypeStruct((_NW, _NCH, _CH, _D), jnp.float32),
        scratch_types=[
            pltpu.VMEM((_NCH, _CH), jnp.int32),
            pltpu.VMEM((_NCH, _CH, _D), jnp.float32),
            pltpu.SemaphoreType.DMA,
        ],
    )
    def gather_k(table_hbm, idx_hbm, out_hbm, idx_v, rows_v, sem):
        wid = lax.axis_index("s") * _NC + lax.axis_index("c")
        pltpu.sync_copy(idx_hbm.at[wid], idx_v)
        copies = [
            pltpu.async_copy(table_hbm.at[idx_v.at[j]], rows_v.at[j], sem)
            for j in range(_NCH)
        ]
        for c in copies:
            c.wait()
        pltpu.sync_copy(rows_v, out_hbm.at[wid])

    return gather_k


def kernel(x, table):
    outs = []
    for k in range(_K):
        idx = _argmax_chunk(x, k)
        idx3 = idx.reshape(_NW, _NCH, _CH)
        outs.append(_gather_sc()(table, idx3).reshape(_BCH, _D))
    return jnp.concatenate(outs, axis=0) if _K > 1 else outs[0]


# speedup vs baseline: 1.1097x; 1.1097x over previous
import functools

import jax
import jax.numpy as jnp
from jax import lax
from jax.experimental import pallas as pl
from jax.experimental.pallas import tpu as pltpu
from jax.experimental.pallas import tpu_sc as plsc

_B = 16384
_N = 1000
_D = 128

_NC = 2
_NS = 16
_NW = _NC * _NS
_BPW = _B // _NW
_CH = 128
_NCH = _BPW // _CH

_BM = 4096
_NSTRIP = 4
_SM = _BM // _NSTRIP


def _argmax_body(*refs):
    strips = refs[:_NSTRIP]
    idx_ref = refs[_NSTRIP]
    parts = [jnp.argmax(s[...], axis=1).astype(jnp.int32) for s in strips]
    idx_ref[...] = jnp.concatenate(parts, axis=0)


def _argmax(x):
    nsteps = _B // _BM
    in_specs = [
        pl.BlockSpec((_SM, _N), lambda i, r=r: (i * _NSTRIP + r, 0))
        for r in range(_NSTRIP)
    ]
    return pl.pallas_call(
        _argmax_body,
        grid=(nsteps,),
        in_specs=in_specs,
        out_specs=pl.BlockSpec((_BM,), lambda i: (i,)),
        out_shape=jax.ShapeDtypeStruct((_B,), jnp.int32),
    )(*([x] * _NSTRIP))


@functools.cache
def _gather_sc():
    mesh = plsc.VectorSubcoreMesh(core_axis_name="c", subcore_axis_name="s")

    @functools.partial(
        pl.kernel,
        mesh=mesh,
        out_type=jax.ShapeDtypeStruct((_NW, _NCH, _CH, _D), jnp.float32),
        scratch_types=[
            pltpu.VMEM((_NCH, _CH), jnp.int32),
            pltpu.VMEM((_NCH, _CH, _D), jnp.float32),
            pltpu.SemaphoreType.DMA,
        ],
    )
    def gather_k(table_hbm, idx_hbm, out_hbm, idx_v, rows_v, sem):
        wid = lax.axis_index("s") * _NC + lax.axis_index("c")
        pltpu.sync_copy(idx_hbm.at[wid], idx_v)
        copies = [
            pltpu.async_copy(table_hbm.at[idx_v.at[j]], rows_v.at[j], sem)
            for j in range(_NCH)
        ]
        for c in copies:
            c.wait()
        pltpu.sync_copy(rows_v, out_hbm.at[wid])

    return gather_k


def kernel(x, table):
    idx = _argmax(x)
    idx3 = idx.reshape(_NW, _NCH, _CH)
    out4 = _gather_sc()(table, idx3)
    return out4.reshape(_B, _D)
